# baseline (speedup 1.0000x reference)
import functools

import jax
import jax.numpy as jnp
from jax import lax
from jax.experimental import pallas as pl
from jax.experimental.pallas import tpu as pltpu
from jax.experimental.pallas import tpu_sc as plsc

E = 8
K = 2
D = 2048
DFF = 768
N = 8192
TB = 1024
NB = N // TB
TILE = 256
L = N * K + E * TILE
NT = L // TILE
NTP = 128
NW = 32
TPW = N // NW
CH_D = 32
CH_C = 16


def _router_body(x_ref, gw_ref, i1_ref, i2_ref, w1_ref, w2_ref):
    x = x_ref[...]
    gw = gw_ref[...]
    logits = lax.dot_general(x, gw, (((1,), (1,)), ((), ())),
                             preferred_element_type=jnp.float32)
    ii = lax.broadcasted_iota(jnp.int32, (TB, E), 1)
    m1 = jnp.max(logits, axis=1, keepdims=True)
    i1 = jnp.min(jnp.where(logits >= m1, ii, E), axis=1, keepdims=True)
    masked = jnp.where(ii == i1, -jnp.inf, logits)
    m2 = jnp.max(masked, axis=1, keepdims=True)
    i2 = jnp.min(jnp.where(masked >= m2, ii, E), axis=1, keepdims=True)
    r = jnp.exp(m2 - m1)
    w1 = 1.0 / (1.0 + r)
    w2 = 1.0 - w1
    i1_ref[0] = i1
    i2_ref[0] = i2
    w1_ref[0] = w1
    w2_ref[0] = w2


def _router(x, gate_w):
    out3 = jax.ShapeDtypeStruct((NB, TB, 1), jnp.int32)
    out3f = jax.ShapeDtypeStruct((NB, TB, 1), jnp.float32)
    return pl.pallas_call(
        _router_body,
        grid=(NB,),
        in_specs=[
            pl.BlockSpec((TB, D), lambda i: (i, 0)),
            pl.BlockSpec((E, D), lambda i: (0, 0)),
        ],
        out_specs=[
            pl.BlockSpec((1, TB, 1), lambda i: (i, 0, 0)),
            pl.BlockSpec((1, TB, 1), lambda i: (i, 0, 0)),
            pl.BlockSpec((1, TB, 1), lambda i: (i, 0, 0)),
            pl.BlockSpec((1, TB, 1), lambda i: (i, 0, 0)),
        ],
        out_shape=[out3, out3, out3f, out3f],
    )(x, gate_w)


def _plan_body(i1_ref, i2_ref, pos1_ref, pos2_ref, ge_ref):
    idx1 = i1_ref[...]
    idx2 = i2_ref[...]
    e3 = lax.broadcasted_iota(jnp.int32, (NB, E, TB), 1)
    sel1 = (idx1.reshape(NB, 1, TB) == e3).astype(jnp.float32)
    sel2 = (idx2.reshape(NB, 1, TB) == e3).astype(jnp.float32)
    cnt = (sel1 + sel2).reshape(NB * E, TB)

    ta = lax.broadcasted_iota(jnp.int32, (TB, TB), 0)
    tb_ = lax.broadcasted_iota(jnp.int32, (TB, TB), 1)
    sl_t = (ta < tb_).astype(jnp.float32)
    excl = lax.dot_general(cnt, sl_t, (((1,), (0,)), ((), ())),
                           preferred_element_type=jnp.float32)

    ones_l = jnp.ones((TB, NTP), jnp.float32)
    s1 = lax.dot_general(cnt, ones_l, (((1,), (0,)), ((), ())),
                         preferred_element_type=jnp.float32)

    i64a = lax.broadcasted_iota(jnp.int32, (NB * E, NB * E), 0)
    i64b = lax.broadcasted_iota(jnp.int32, (NB * E, NB * E), 1)
    r_i, e_i = i64a // E, i64a % E
    r_j, e_j = i64b // E, i64b % E
    hi = lax.Precision.HIGHEST
    m_roff = ((r_j < r_i) & (e_j == e_i)).astype(jnp.float32)
    roff = lax.dot_general(m_roff, s1, (((1,), (0,)), ((), ())),
                           precision=hi,
                           preferred_element_type=jnp.float32)
    m_tot = (e_j == e_i).astype(jnp.float32)
    tot = lax.dot_general(m_tot, s1, (((1,), (0,)), ((), ())),
                          precision=hi,
                          preferred_element_type=jnp.float32)
    pc = jnp.floor((tot + (TILE - 1.0)) * (1.0 / TILE)) * TILE
    m_start = ((e_j < e_i) & (r_j == 0)).astype(jnp.float32)
    start = lax.dot_general(m_start, pc, (((1,), (0,)), ((), ())),
                            precision=hi,
                            preferred_element_type=jnp.float32)

    base = (excl + (roff + start)[:, :1]).reshape(NB, E, TB)
    pos1 = jnp.sum(sel1 * base, axis=1)
    pos2 = jnp.sum(sel2 * base, axis=1)
    pos1_ref[...] = pos1.astype(jnp.int32)
    pos2_ref[...] = pos2.astype(jnp.int32)

    start8 = start[:E, :1]
    pc8 = pc[:E, :1]
    tbase = lax.broadcasted_iota(jnp.int32, (E, NTP), 1).astype(jnp.float32) * TILE
    ind = ((tbase >= start8) & (tbase < start8 + pc8)).astype(jnp.float32)
    e_rows = lax.broadcasted_iota(jnp.int32, (E, NTP), 0).astype(jnp.float32)
    ge = jnp.sum(ind * e_rows, axis=0, keepdims=True)
    ge_ref[...] = ge.astype(jnp.int32)


def _plan(idx1, idx2):
    return pl.pallas_call(
        _plan_body,
        out_shape=[
            jax.ShapeDtypeStruct((NB, TB), jnp.int32),
            jax.ShapeDtypeStruct((NB, TB), jnp.int32),
            jax.ShapeDtypeStruct((1, NTP), jnp.int32),
        ],
    )(idx1, idx2)


@functools.lru_cache(maxsize=None)
def _make_dispatch():
    mesh = plsc.VectorSubcoreMesh(core_axis_name="c", subcore_axis_name="s")

    @functools.partial(
        pl.kernel,
        mesh=mesh,
        out_type=jax.ShapeDtypeStruct((L, D), jnp.float32),
        scratch_types=[
            pltpu.VMEM((CH_D, D), jnp.float32),
            pltpu.VMEM((CH_D,), jnp.int32),
            pltpu.VMEM((CH_D,), jnp.int32),
            pltpu.SemaphoreType.DMA,
        ],
    )
    def dispatch(x_hbm, pos1_hbm, pos2_hbm, xs_hbm, rows_v, i1_v, i2_v, sem):
        wid = lax.axis_index("s") * 2 + lax.axis_index("c")

        def body(c, _):
            base = wid * TPW + c * CH_D
            pltpu.sync_copy(x_hbm.at[pl.ds(base, CH_D)], rows_v)
            pltpu.sync_copy(pos1_hbm.at[pl.ds(base, CH_D)], i1_v)
            pltpu.sync_copy(pos2_hbm.at[pl.ds(base, CH_D)], i2_v)
            cp1 = pltpu.async_copy(rows_v, xs_hbm.at[i1_v], sem)
            cp2 = pltpu.async_copy(rows_v, xs_hbm.at[i2_v], sem)
            cp1.wait()
            cp2.wait()
            return 0

        lax.fori_loop(0, TPW // CH_D, body, 0)

    return dispatch


def _dispatch(x, pos1f, pos2f):
    return _make_dispatch()(x, pos1f, pos2f)


def _mm_body(ge_ref, xs_ref, wg_ref, wu_ref, wd_ref, ys_ref,
             wgb, wub, wdb, st):
    i = pl.program_id(0)
    e = ge_ref[i]

    @pl.when(jnp.logical_or(i == 0, st[0] != e))
    def _():
        wgb[...] = wg_ref[0].astype(jnp.bfloat16)
        wub[...] = wu_ref[0].astype(jnp.bfloat16)
        wdb[...] = wd_ref[0].astype(jnp.bfloat16)
        st[0] = e

    x = xs_ref[...].astype(jnp.bfloat16)
    g = lax.dot_general(x, wgb[...], (((1,), (1,)), ((), ())),
                        preferred_element_type=jnp.float32)
    u = lax.dot_general(x, wub[...], (((1,), (1,)), ((), ())),
                        preferred_element_type=jnp.float32)
    h = (g * jax.nn.sigmoid(g) * u).astype(jnp.bfloat16)
    ys_ref[...] = lax.dot_general(h, wdb[...], (((1,), (1,)), ((), ())),
                                  preferred_element_type=jnp.float32)


def _mm(ge, xs, w_gate, w_up, w_down):
    return pl.pallas_call(
        _mm_body,
        grid_spec=pltpu.PrefetchScalarGridSpec(
            num_scalar_prefetch=1,
            grid=(NT,),
            in_specs=[
                pl.BlockSpec((TILE, D), lambda i, ge_s: (i, 0)),
                pl.BlockSpec((1, DFF, D), lambda i, ge_s: (ge_s[i], 0, 0)),
                pl.BlockSpec((1, DFF, D), lambda i, ge_s: (ge_s[i], 0, 0)),
                pl.BlockSpec((1, D, DFF), lambda i, ge_s: (ge_s[i], 0, 0)),
            ],
            out_specs=pl.BlockSpec((TILE, D), lambda i, ge_s: (i, 0)),
            scratch_shapes=[
                pltpu.VMEM((DFF, D), jnp.bfloat16),
                pltpu.VMEM((DFF, D), jnp.bfloat16),
                pltpu.VMEM((D, DFF), jnp.bfloat16),
                pltpu.SMEM((1,), jnp.int32),
            ],
        ),
        out_shape=jax.ShapeDtypeStruct((L, D), jnp.float32),
    )(ge, xs, w_gate, w_up, w_down)


@functools.lru_cache(maxsize=None)
def _make_combine():
    mesh = plsc.VectorSubcoreMesh(core_axis_name="c", subcore_axis_name="s")

    @functools.partial(
        pl.kernel,
        mesh=mesh,
        compiler_params=pltpu.CompilerParams(needs_layout_passes=False),
        out_type=jax.ShapeDtypeStruct((N, D), jnp.float32),
        scratch_types=[
            pltpu.VMEM((CH_C, D), jnp.float32),
            pltpu.VMEM((CH_C, D), jnp.float32),
            pltpu.VMEM((CH_C,), jnp.int32),
            pltpu.VMEM((CH_C,), jnp.int32),
            pltpu.VMEM((CH_C,), jnp.float32),
            pltpu.VMEM((CH_C,), jnp.float32),
            pltpu.SemaphoreType.DMA,
        ],
    )
    def combine(ys_hbm, pos1_hbm, pos2_hbm, w1_hbm, w2_hbm, out_hbm,
                r1_v, r2_v, i1_v, i2_v, w1_v, w2_v, sem):
        wid = lax.axis_index("s") * 2 + lax.axis_index("c")

        def body(c, _):
            base = wid * TPW + c * CH_C
            pltpu.sync_copy(pos1_hbm.at[pl.ds(base, CH_C)], i1_v)
            pltpu.sync_copy(pos2_hbm.at[pl.ds(base, CH_C)], i2_v)
            pltpu.sync_copy(w1_hbm.at[pl.ds(base, CH_C)], w1_v)
            pltpu.sync_copy(w2_hbm.at[pl.ds(base, CH_C)], w2_v)
            cp1 = pltpu.async_copy(ys_hbm.at[i1_v], r1_v, sem)
            cp2 = pltpu.async_copy(ys_hbm.at[i2_v], r2_v, sem)
            cp1.wait()
            cp2.wait()
            lane = lax.broadcasted_iota(jnp.int32, (16,), 0)
            wv1 = w1_v[...]
            wv2 = w2_v[...]
            for j in range(CH_C):
                sel = (lane == j).astype(jnp.float32)
                a1 = jnp.full((16,), jnp.sum(wv1 * sel, axis=0), jnp.float32)
                a2 = jnp.full((16,), jnp.sum(wv2 * sel, axis=0), jnp.float32)

                def col(cc, _):
                    for u in range(8):
                        sl = pl.ds(cc * 128 + u * 16, 16)
                        r1_v[j, sl] = a1 * r1_v[j, sl] + a2 * r2_v[j, sl]
                    return 0

                lax.fori_loop(0, D // 128, col, 0)
            pltpu.sync_copy(r1_v, out_hbm.at[pl.ds(base, CH_C)])
            return 0

        lax.fori_loop(0, TPW // CH_C, body, 0)

    return combine


def _combine(ys, pos1f, pos2f, w1f, w2f):
    return _make_combine()(ys, pos1f, pos2f, w1f, w2f)


def kernel(hidden_states, gate_w, w_gate, w_up, w_down):
    b, s, d = hidden_states.shape
    x = hidden_states.reshape(b * s, d)
    i1o, i2o, w1o, w2o = _router(x, gate_w)
    pos1, pos2, ge = _plan(i1o.reshape(NB, TB), i2o.reshape(NB, TB))
    pos1f = pos1.reshape(N)
    pos2f = pos2.reshape(N)
    xs = _dispatch(x, pos1f, pos2f)
    ys = _mm(ge.reshape(NTP), xs, w_gate, w_up, w_down)
    out = _combine(ys, pos1f, pos2f, w1o.reshape(N), w2o.reshape(N))
    return out.reshape(b, s, d)


# speedup vs baseline: 1.1592x; 1.1592x over previous
import functools

import jax
import jax.numpy as jnp
from jax import lax
from jax.experimental import pallas as pl
from jax.experimental.pallas import tpu as pltpu
from jax.experimental.pallas import tpu_sc as plsc

E = 8
K = 2
D = 2048
DFF = 768
N = 8192
TB = 1024
NB = N // TB
TILE = 256
L = N * K + E * TILE
NT = L // TILE
NTP = 128
NW = 32
TPW = N // NW
CH_D = 32
CH_C = 8


def _router_body(x_ref, gw_ref, i1_ref, i2_ref, w1_ref, w2_ref):
    x = x_ref[...]
    gw = gw_ref[...]
    logits = lax.dot_general(x, gw, (((1,), (1,)), ((), ())),
                             preferred_element_type=jnp.float32)
    ii = lax.broadcasted_iota(jnp.int32, (TB, E), 1)
    m1 = jnp.max(logits, axis=1, keepdims=True)
    i1 = jnp.min(jnp.where(logits >= m1, ii, E), axis=1, keepdims=True)
    masked = jnp.where(ii == i1, -jnp.inf, logits)
    m2 = jnp.max(masked, axis=1, keepdims=True)
    i2 = jnp.min(jnp.where(masked >= m2, ii, E), axis=1, keepdims=True)
    r = jnp.exp(m2 - m1)
    w1 = 1.0 / (1.0 + r)
    w2 = 1.0 - w1
    i1_ref[0] = i1
    i2_ref[0] = i2
    w1_ref[0] = w1
    w2_ref[0] = w2


def _router(x, gate_w):
    out3 = jax.ShapeDtypeStruct((NB, TB, 1), jnp.int32)
    out3f = jax.ShapeDtypeStruct((NB, TB, 1), jnp.float32)
    return pl.pallas_call(
        _router_body,
        grid=(NB,),
        in_specs=[
            pl.BlockSpec((TB, D), lambda i: (i, 0)),
            pl.BlockSpec((E, D), lambda i: (0, 0)),
        ],
        out_specs=[
            pl.BlockSpec((1, TB, 1), lambda i: (i, 0, 0)),
            pl.BlockSpec((1, TB, 1), lambda i: (i, 0, 0)),
            pl.BlockSpec((1, TB, 1), lambda i: (i, 0, 0)),
            pl.BlockSpec((1, TB, 1), lambda i: (i, 0, 0)),
        ],
        out_shape=[out3, out3, out3f, out3f],
    )(x, gate_w)


def _plan_body(i1_ref, i2_ref, pos1_ref, pos2_ref, ge_ref):
    idx1 = i1_ref[...]
    idx2 = i2_ref[...]
    e3 = lax.broadcasted_iota(jnp.int32, (NB, E, TB), 1)
    sel1 = (idx1.reshape(NB, 1, TB) == e3).astype(jnp.float32)
    sel2 = (idx2.reshape(NB, 1, TB) == e3).astype(jnp.float32)
    cnt = (sel1 + sel2).reshape(NB * E, TB)

    ta = lax.broadcasted_iota(jnp.int32, (TB, TB), 0)
    tb_ = lax.broadcasted_iota(jnp.int32, (TB, TB), 1)
    sl_t = (ta < tb_).astype(jnp.float32)
    excl = lax.dot_general(cnt, sl_t, (((1,), (0,)), ((), ())),
                           preferred_element_type=jnp.float32)

    ones_l = jnp.ones((TB, NTP), jnp.float32)
    s1 = lax.dot_general(cnt, ones_l, (((1,), (0,)), ((), ())),
                         preferred_element_type=jnp.float32)

    i64a = lax.broadcasted_iota(jnp.int32, (NB * E, NB * E), 0)
    i64b = lax.broadcasted_iota(jnp.int32, (NB * E, NB * E), 1)
    r_i, e_i = i64a // E, i64a % E
    r_j, e_j = i64b // E, i64b % E
    hi = lax.Precision.HIGHEST
    m_roff = ((r_j < r_i) & (e_j == e_i)).astype(jnp.float32)
    roff = lax.dot_general(m_roff, s1, (((1,), (0,)), ((), ())),
                           precision=hi,
                           preferred_element_type=jnp.float32)
    m_tot = (e_j == e_i).astype(jnp.float32)
    tot = lax.dot_general(m_tot, s1, (((1,), (0,)), ((), ())),
                          precision=hi,
                          preferred_element_type=jnp.float32)
    pc = jnp.floor((tot + (TILE - 1.0)) * (1.0 / TILE)) * TILE
    m_start = ((e_j < e_i) & (r_j == 0)).astype(jnp.float32)
    start = lax.dot_general(m_start, pc, (((1,), (0,)), ((), ())),
                            precision=hi,
                            preferred_element_type=jnp.float32)

    base = (excl + (roff + start)[:, :1]).reshape(NB, E, TB)
    pos1 = jnp.sum(sel1 * base, axis=1)
    pos2 = jnp.sum(sel2 * base, axis=1)
    pos1_ref[...] = pos1.astype(jnp.int32)
    pos2_ref[...] = pos2.astype(jnp.int32)

    start8 = start[:E, :1]
    pc8 = pc[:E, :1]
    tbase = lax.broadcasted_iota(jnp.int32, (E, NTP), 1).astype(jnp.float32) * TILE
    ind = ((tbase >= start8) & (tbase < start8 + pc8)).astype(jnp.float32)
    e_rows = lax.broadcasted_iota(jnp.int32, (E, NTP), 0).astype(jnp.float32)
    ge = jnp.sum(ind * e_rows, axis=0, keepdims=True)
    ge_ref[...] = ge.astype(jnp.int32)


def _plan(idx1, idx2):
    return pl.pallas_call(
        _plan_body,
        out_shape=[
            jax.ShapeDtypeStruct((NB, TB), jnp.int32),
            jax.ShapeDtypeStruct((NB, TB), jnp.int32),
            jax.ShapeDtypeStruct((1, NTP), jnp.int32),
        ],
    )(idx1, idx2)


@functools.lru_cache(maxsize=None)
def _make_dispatch():
    mesh = plsc.VectorSubcoreMesh(core_axis_name="c", subcore_axis_name="s")

    @functools.partial(
        pl.kernel,
        mesh=mesh,
        out_type=jax.ShapeDtypeStruct((L, D), jnp.float32),
        scratch_types=[
            pltpu.VMEM((CH_D, D), jnp.float32),
            pltpu.VMEM((CH_D,), jnp.int32),
            pltpu.VMEM((CH_D,), jnp.int32),
            pltpu.SemaphoreType.DMA,
        ],
    )
    def dispatch(x_hbm, pos1_hbm, pos2_hbm, xs_hbm, rows_v, i1_v, i2_v, sem):
        wid = lax.axis_index("s") * 2 + lax.axis_index("c")

        def body(c, _):
            base = wid * TPW + c * CH_D
            pltpu.sync_copy(x_hbm.at[pl.ds(base, CH_D)], rows_v)
            pltpu.sync_copy(pos1_hbm.at[pl.ds(base, CH_D)], i1_v)
            pltpu.sync_copy(pos2_hbm.at[pl.ds(base, CH_D)], i2_v)
            cp1 = pltpu.async_copy(rows_v, xs_hbm.at[i1_v], sem)
            cp2 = pltpu.async_copy(rows_v, xs_hbm.at[i2_v], sem)
            cp1.wait()
            cp2.wait()
            return 0

        lax.fori_loop(0, TPW // CH_D, body, 0)

    return dispatch


def _dispatch(x, pos1f, pos2f):
    return _make_dispatch()(x, pos1f, pos2f)


def _mm_body(ge_ref, xs_ref, wg_ref, wu_ref, wd_ref, ys_ref):
    x = xs_ref[...]
    g = lax.dot_general(x, wg_ref[0], (((1,), (1,)), ((), ())),
                        preferred_element_type=jnp.float32)
    u = lax.dot_general(x, wu_ref[0], (((1,), (1,)), ((), ())),
                        preferred_element_type=jnp.float32)
    h = g * jax.nn.sigmoid(g) * u
    ys_ref[...] = lax.dot_general(h, wd_ref[0], (((1,), (1,)), ((), ())),
                                  preferred_element_type=jnp.float32)


def _mm(ge, xs, w_gate, w_up, w_down):
    return pl.pallas_call(
        _mm_body,
        grid_spec=pltpu.PrefetchScalarGridSpec(
            num_scalar_prefetch=1,
            grid=(NT,),
            in_specs=[
                pl.BlockSpec((TILE, D), lambda i, ge_s: (i, 0)),
                pl.BlockSpec((1, DFF, D), lambda i, ge_s: (ge_s[i], 0, 0)),
                pl.BlockSpec((1, DFF, D), lambda i, ge_s: (ge_s[i], 0, 0)),
                pl.BlockSpec((1, D, DFF), lambda i, ge_s: (ge_s[i], 0, 0)),
            ],
            out_specs=pl.BlockSpec((TILE, D), lambda i, ge_s: (i, 0)),
        ),
        out_shape=jax.ShapeDtypeStruct((L, D), jnp.float32),
    )(ge, xs, w_gate, w_up, w_down)


@functools.lru_cache(maxsize=None)
def _make_combine():
    mesh = plsc.VectorSubcoreMesh(core_axis_name="c", subcore_axis_name="s")

    @functools.partial(
        pl.kernel,
        mesh=mesh,
        compiler_params=pltpu.CompilerParams(needs_layout_passes=False),
        out_type=jax.ShapeDtypeStruct((N, D), jnp.float32),
        scratch_types=[
            pltpu.VMEM((CH_C, D), jnp.float32),
            pltpu.VMEM((CH_C, D), jnp.float32),
            pltpu.VMEM((CH_C, D), jnp.float32),
            pltpu.VMEM((CH_C, D), jnp.float32),
            pltpu.VMEM((TPW,), jnp.int32),
            pltpu.VMEM((TPW,), jnp.int32),
            pltpu.VMEM((TPW,), jnp.float32),
            pltpu.VMEM((TPW,), jnp.float32),
            pltpu.SemaphoreType.DMA,
            pltpu.SemaphoreType.DMA,
            pltpu.SemaphoreType.DMA,
            pltpu.SemaphoreType.DMA,
        ],
    )
    def combine(ys_hbm, pos1_hbm, pos2_hbm, w1_hbm, w2_hbm, out_hbm,
                r1a, r2a, r1b, r2b, i1_v, i2_v, w1_v, w2_v,
                sga, sgb, ssa, ssb):
        wid = lax.axis_index("s") * 2 + lax.axis_index("c")
        base0 = wid * TPW
        nch = TPW // CH_C
        npair = nch // 2

        pltpu.sync_copy(pos1_hbm.at[pl.ds(base0, TPW)], i1_v)
        pltpu.sync_copy(pos2_hbm.at[pl.ds(base0, TPW)], i2_v)
        pltpu.sync_copy(w1_hbm.at[pl.ds(base0, TPW)], w1_v)
        pltpu.sync_copy(w2_hbm.at[pl.ds(base0, TPW)], w2_v)

        slots = [(r1a, r2a, sga, ssa), (r1b, r2b, sgb, ssb)]

        def issue(c, s):
            r1s, r2s, sg, _ = slots[s]
            pltpu.async_copy(ys_hbm.at[i1_v.at[pl.ds(c * CH_C, CH_C)]],
                             r1s, sg)
            pltpu.async_copy(ys_hbm.at[i2_v.at[pl.ds(c * CH_C, CH_C)]],
                             r2s, sg)

        def drain_gather(c, s):
            r1s, r2s, sg, _ = slots[s]
            pltpu.make_async_copy(ys_hbm.at[i1_v.at[pl.ds(c * CH_C, CH_C)]],
                                  r1s, sg).wait()
            pltpu.make_async_copy(ys_hbm.at[i2_v.at[pl.ds(c * CH_C, CH_C)]],
                                  r2s, sg).wait()

        def drain_store(c, s):
            r1s, _, _, ss = slots[s]
            pltpu.make_async_copy(r1s, out_hbm.at[pl.ds(base0 + c * CH_C,
                                                        CH_C)], ss).wait()

        lane = lax.broadcasted_iota(jnp.int32, (16,), 0)

        def compute(c, s, wv1, wv2, lane_off):
            r1s, _, _, ss = slots[s]
            r2s = slots[s][1]
            for j in range(CH_C):
                sel = (lane == lane_off + j).astype(jnp.float32)
                a1 = jnp.full((16,), jnp.sum(wv1 * sel, axis=0), jnp.float32)
                a2 = jnp.full((16,), jnp.sum(wv2 * sel, axis=0), jnp.float32)

                def col(cc, _):
                    for u in range(8):
                        sl = pl.ds(cc * 128 + u * 16, 16)
                        r1s[j, sl] = a1 * r1s[j, sl] + a2 * r2s[j, sl]
                    return 0

                lax.fori_loop(0, D // 128, col, 0)
            pltpu.async_copy(r1s, out_hbm.at[pl.ds(base0 + c * CH_C, CH_C)],
                             ss)

        issue(0, 0)

        def body(p, _):
            c0 = 2 * p
            c1 = c0 + 1
            issue(c1, 1)
            wv1 = w1_v[pl.ds(p * 16, 16)]
            wv2 = w2_v[pl.ds(p * 16, 16)]

            @pl.when(p > 0)
            def _():
                drain_store(c0 - 2, 0)

            drain_gather(c0, 0)
            compute(c0, 0, wv1, wv2, 0)

            @pl.when(c0 + 2 < nch)
            def _():
                issue(c0 + 2, 0)

            @pl.when(p > 0)
            def _():
                drain_store(c1 - 2, 1)

            drain_gather(c1, 1)
            compute(c1, 1, wv1, wv2, CH_C)
            return 0

        lax.fori_loop(0, npair, body, 0)
        drain_store(nch - 2, 0)
        drain_store(nch - 1, 1)

    return combine


def _combine(ys, pos1f, pos2f, w1f, w2f):
    return _make_combine()(ys, pos1f, pos2f, w1f, w2f)


def kernel(hidden_states, gate_w, w_gate, w_up, w_down):
    b, s, d = hidden_states.shape
    x = hidden_states.reshape(b * s, d)
    i1o, i2o, w1o, w2o = _router(x, gate_w)
    pos1, pos2, ge = _plan(i1o.reshape(NB, TB), i2o.reshape(NB, TB))
    pos1f = pos1.reshape(N)
    pos2f = pos2.reshape(N)
    xs = _dispatch(x, pos1f, pos2f)
    ys = _mm(ge.reshape(NTP), xs, w_gate, w_up, w_down)
    out = _combine(ys, pos1f, pos2f, w1o.reshape(N), w2o.reshape(N))
    return out.reshape(b, s, d)


# speedup vs baseline: 1.1749x; 1.0135x over previous
import functools

import jax
import jax.numpy as jnp
from jax import lax
from jax.experimental import pallas as pl
from jax.experimental.pallas import tpu as pltpu
from jax.experimental.pallas import tpu_sc as plsc

E = 8
K = 2
D = 2048
DFF = 768
N = 8192
TB = 1024
NB = N // TB
TILE = 256
L = N * K + E * TILE
NT = L // TILE
NTP = 128
NW = 32
TPW = N // NW
CH_D = 16
CH_C = 8


def _router_body(x_ref, gw_ref, i1_ref, i2_ref, w1_ref, w2_ref):
    x = x_ref[...]
    gw = gw_ref[...]
    logits = lax.dot_general(x, gw, (((1,), (1,)), ((), ())),
                             preferred_element_type=jnp.float32)
    ii = lax.broadcasted_iota(jnp.int32, (TB, E), 1)
    m1 = jnp.max(logits, axis=1, keepdims=True)
    i1 = jnp.min(jnp.where(logits >= m1, ii, E), axis=1, keepdims=True)
    masked = jnp.where(ii == i1, -jnp.inf, logits)
    m2 = jnp.max(masked, axis=1, keepdims=True)
    i2 = jnp.min(jnp.where(masked >= m2, ii, E), axis=1, keepdims=True)
    r = jnp.exp(m2 - m1)
    w1 = 1.0 / (1.0 + r)
    w2 = 1.0 - w1
    i1_ref[0] = i1
    i2_ref[0] = i2
    w1_ref[0] = w1
    w2_ref[0] = w2


def _router(x, gate_w):
    out3 = jax.ShapeDtypeStruct((NB, TB, 1), jnp.int32)
    out3f = jax.ShapeDtypeStruct((NB, TB, 1), jnp.float32)
    return pl.pallas_call(
        _router_body,
        grid=(NB,),
        in_specs=[
            pl.BlockSpec((TB, D), lambda i: (i, 0)),
            pl.BlockSpec((E, D), lambda i: (0, 0)),
        ],
        out_specs=[
            pl.BlockSpec((1, TB, 1), lambda i: (i, 0, 0)),
            pl.BlockSpec((1, TB, 1), lambda i: (i, 0, 0)),
            pl.BlockSpec((1, TB, 1), lambda i: (i, 0, 0)),
            pl.BlockSpec((1, TB, 1), lambda i: (i, 0, 0)),
        ],
        out_shape=[out3, out3, out3f, out3f],
    )(x, gate_w)


def _plan_body(i1_ref, i2_ref, pos1_ref, pos2_ref, ge_ref):
    idx1 = i1_ref[...]
    idx2 = i2_ref[...]
    e3 = lax.broadcasted_iota(jnp.int32, (NB, E, TB), 1)
    sel1 = (idx1.reshape(NB, 1, TB) == e3).astype(jnp.float32)
    sel2 = (idx2.reshape(NB, 1, TB) == e3).astype(jnp.float32)
    cnt = (sel1 + sel2).reshape(NB * E, TB)

    ta = lax.broadcasted_iota(jnp.int32, (TB, TB), 0)
    tb_ = lax.broadcasted_iota(jnp.int32, (TB, TB), 1)
    sl_t = (ta < tb_).astype(jnp.float32)
    excl = lax.dot_general(cnt, sl_t, (((1,), (0,)), ((), ())),
                           preferred_element_type=jnp.float32)

    ones_l = jnp.ones((TB, NTP), jnp.float32)
    s1 = lax.dot_general(cnt, ones_l, (((1,), (0,)), ((), ())),
                         preferred_element_type=jnp.float32)

    i64a = lax.broadcasted_iota(jnp.int32, (NB * E, NB * E), 0)
    i64b = lax.broadcasted_iota(jnp.int32, (NB * E, NB * E), 1)
    r_i, e_i = i64a // E, i64a % E
    r_j, e_j = i64b // E, i64b % E
    hi = lax.Precision.HIGHEST
    m_roff = ((r_j < r_i) & (e_j == e_i)).astype(jnp.float32)
    roff = lax.dot_general(m_roff, s1, (((1,), (0,)), ((), ())),
                           precision=hi,
                           preferred_element_type=jnp.float32)
    m_tot = (e_j == e_i).astype(jnp.float32)
    tot = lax.dot_general(m_tot, s1, (((1,), (0,)), ((), ())),
                          precision=hi,
                          preferred_element_type=jnp.float32)
    pc = jnp.floor((tot + (TILE - 1.0)) * (1.0 / TILE)) * TILE
    m_start = ((e_j < e_i) & (r_j == 0)).astype(jnp.float32)
    start = lax.dot_general(m_start, pc, (((1,), (0,)), ((), ())),
                            precision=hi,
                            preferred_element_type=jnp.float32)

    base = (excl + (roff + start)[:, :1]).reshape(NB, E, TB)
    pos1 = jnp.sum(sel1 * base, axis=1)
    pos2 = jnp.sum(sel2 * base, axis=1)
    pos1_ref[...] = pos1.astype(jnp.int32)
    pos2_ref[...] = pos2.astype(jnp.int32)

    start8 = start[:E, :1]
    pc8 = pc[:E, :1]
    tbase = lax.broadcasted_iota(jnp.int32, (E, NTP), 1).astype(jnp.float32) * TILE
    ind = ((tbase >= start8) & (tbase < start8 + pc8)).astype(jnp.float32)
    e_rows = lax.broadcasted_iota(jnp.int32, (E, NTP), 0).astype(jnp.float32)
    ge = jnp.sum(ind * e_rows, axis=0, keepdims=True)
    ge_ref[...] = ge.astype(jnp.int32)


def _plan(idx1, idx2):
    return pl.pallas_call(
        _plan_body,
        out_shape=[
            jax.ShapeDtypeStruct((NB, TB), jnp.int32),
            jax.ShapeDtypeStruct((NB, TB), jnp.int32),
            jax.ShapeDtypeStruct((1, NTP), jnp.int32),
        ],
    )(idx1, idx2)


@functools.lru_cache(maxsize=None)
def _make_dispatch():
    mesh = plsc.VectorSubcoreMesh(core_axis_name="c", subcore_axis_name="s")

    @functools.partial(
        pl.kernel,
        mesh=mesh,
        out_type=jax.ShapeDtypeStruct((L, D), jnp.float32),
        scratch_types=[
            pltpu.VMEM((CH_D, D), jnp.float32),
            pltpu.VMEM((CH_D, D), jnp.float32),
            pltpu.VMEM((CH_D,), jnp.int32),
            pltpu.VMEM((CH_D,), jnp.int32),
            pltpu.VMEM((CH_D,), jnp.int32),
            pltpu.VMEM((CH_D,), jnp.int32),
            pltpu.SemaphoreType.DMA,
            pltpu.SemaphoreType.DMA,
            pltpu.SemaphoreType.DMA,
            pltpu.SemaphoreType.DMA,
        ],
    )
    def dispatch(x_hbm, pos1_hbm, pos2_hbm, xs_hbm,
                 rowsa, rowsb, i1a, i2a, i1b, i2b, sla, slb, sca, scb):
        wid = lax.axis_index("s") * 2 + lax.axis_index("c")
        base0 = wid * TPW
        nch = TPW // CH_D

        slots = [(rowsa, i1a, i2a, sla, sca), (rowsb, i1b, i2b, slb, scb)]

        def load_issue(c, s):
            rows, i1s, i2s, sl, _ = slots[s]
            base = base0 + c * CH_D
            pltpu.sync_copy(pos1_hbm.at[pl.ds(base, CH_D)], i1s)
            pltpu.sync_copy(pos2_hbm.at[pl.ds(base, CH_D)], i2s)
            pltpu.async_copy(x_hbm.at[pl.ds(base, CH_D)], rows, sl)

        def load_wait(c, s):
            rows, _, _, sl, _ = slots[s]
            base = base0 + c * CH_D
            pltpu.make_async_copy(x_hbm.at[pl.ds(base, CH_D)], rows,
                                  sl).wait()

        def scatter_issue(s):
            rows, i1s, i2s, _, sc = slots[s]
            pltpu.async_copy(rows, xs_hbm.at[i1s], sc)
            pltpu.async_copy(rows, xs_hbm.at[i2s], sc)

        def scatter_drain(s):
            rows, i1s, i2s, _, sc = slots[s]
            pltpu.make_async_copy(rows, xs_hbm.at[i1s], sc).wait()
            pltpu.make_async_copy(rows, xs_hbm.at[i2s], sc).wait()

        load_issue(0, 0)
        load_issue(1, 1)

        def body(p, _):
            c0 = 2 * p
            c1 = c0 + 1
            load_wait(c0, 0)
            scatter_issue(0)
            load_wait(c1, 1)
            scatter_issue(1)

            @pl.when(c0 + 2 < nch)
            def _():
                scatter_drain(0)
                load_issue(c0 + 2, 0)

            @pl.when(c1 + 2 < nch)
            def _():
                scatter_drain(1)
                load_issue(c1 + 2, 1)

            return 0

        lax.fori_loop(0, nch // 2, body, 0)
        scatter_drain(0)
        scatter_drain(1)

    return dispatch


def _dispatch(x, pos1f, pos2f):
    return _make_dispatch()(x, pos1f, pos2f)


def _mm_body(ge_ref, xs_ref, wg_ref, wu_ref, wd_ref, ys_ref):
    x = xs_ref[...]
    g = lax.dot_general(x, wg_ref[0], (((1,), (1,)), ((), ())),
                        preferred_element_type=jnp.float32)
    u = lax.dot_general(x, wu_ref[0], (((1,), (1,)), ((), ())),
                        preferred_element_type=jnp.float32)
    h = g * jax.nn.sigmoid(g) * u
    ys_ref[...] = lax.dot_general(h, wd_ref[0], (((1,), (1,)), ((), ())),
                                  preferred_element_type=jnp.float32)


def _mm(ge, xs, w_gate, w_up, w_down):
    return pl.pallas_call(
        _mm_body,
        grid_spec=pltpu.PrefetchScalarGridSpec(
            num_scalar_prefetch=1,
            grid=(NT,),
            in_specs=[
                pl.BlockSpec((TILE, D), lambda i, ge_s: (i, 0)),
                pl.BlockSpec((1, DFF, D), lambda i, ge_s: (ge_s[i], 0, 0)),
                pl.BlockSpec((1, DFF, D), lambda i, ge_s: (ge_s[i], 0, 0)),
                pl.BlockSpec((1, D, DFF), lambda i, ge_s: (ge_s[i], 0, 0)),
            ],
            out_specs=pl.BlockSpec((TILE, D), lambda i, ge_s: (i, 0)),
        ),
        out_shape=jax.ShapeDtypeStruct((L, D), jnp.float32),
    )(ge, xs, w_gate, w_up, w_down)


@functools.lru_cache(maxsize=None)
def _make_combine():
    mesh = plsc.VectorSubcoreMesh(core_axis_name="c", subcore_axis_name="s")

    @functools.partial(
        pl.kernel,
        mesh=mesh,
        compiler_params=pltpu.CompilerParams(needs_layout_passes=False),
        out_type=jax.ShapeDtypeStruct((N, D), jnp.float32),
        scratch_types=[
            pltpu.VMEM((CH_C, D), jnp.float32),
            pltpu.VMEM((CH_C, D), jnp.float32),
            pltpu.VMEM((CH_C, D), jnp.float32),
            pltpu.VMEM((CH_C, D), jnp.float32),
            pltpu.VMEM((TPW,), jnp.int32),
            pltpu.VMEM((TPW,), jnp.int32),
            pltpu.VMEM((TPW,), jnp.float32),
            pltpu.VMEM((TPW,), jnp.float32),
            pltpu.SemaphoreType.DMA,
            pltpu.SemaphoreType.DMA,
            pltpu.SemaphoreType.DMA,
            pltpu.SemaphoreType.DMA,
        ],
    )
    def combine(ys_hbm, pos1_hbm, pos2_hbm, w1_hbm, w2_hbm, out_hbm,
                r1a, r2a, r1b, r2b, i1_v, i2_v, w1_v, w2_v,
                sga, sgb, ssa, ssb):
        wid = lax.axis_index("s") * 2 + lax.axis_index("c")
        base0 = wid * TPW
        nch = TPW // CH_C
        npair = nch // 2

        pltpu.sync_copy(pos1_hbm.at[pl.ds(base0, TPW)], i1_v)
        pltpu.sync_copy(pos2_hbm.at[pl.ds(base0, TPW)], i2_v)
        pltpu.sync_copy(w1_hbm.at[pl.ds(base0, TPW)], w1_v)
        pltpu.sync_copy(w2_hbm.at[pl.ds(base0, TPW)], w2_v)

        slots = [(r1a, r2a, sga, ssa), (r1b, r2b, sgb, ssb)]

        def issue(c, s):
            r1s, r2s, sg, _ = slots[s]
            pltpu.async_copy(ys_hbm.at[i1_v.at[pl.ds(c * CH_C, CH_C)]],
                             r1s, sg)
            pltpu.async_copy(ys_hbm.at[i2_v.at[pl.ds(c * CH_C, CH_C)]],
                             r2s, sg)

        def drain_gather(c, s):
            r1s, r2s, sg, _ = slots[s]
            pltpu.make_async_copy(ys_hbm.at[i1_v.at[pl.ds(c * CH_C, CH_C)]],
                                  r1s, sg).wait()
            pltpu.make_async_copy(ys_hbm.at[i2_v.at[pl.ds(c * CH_C, CH_C)]],
                                  r2s, sg).wait()

        def drain_store(c, s):
            r1s, _, _, ss = slots[s]
            pltpu.make_async_copy(r1s, out_hbm.at[pl.ds(base0 + c * CH_C,
                                                        CH_C)], ss).wait()

        lane = lax.broadcasted_iota(jnp.int32, (16,), 0)

        def compute(c, s, wv1, wv2, lane_off):
            r1s, _, _, ss = slots[s]
            r2s = slots[s][1]
            for j in range(CH_C):
                sel = (lane == lane_off + j).astype(jnp.float32)
                a1 = jnp.full((16,), jnp.sum(wv1 * sel, axis=0), jnp.float32)
                a2 = jnp.full((16,), jnp.sum(wv2 * sel, axis=0), jnp.float32)

                def col(cc, _):
                    for u in range(8):
                        sl = pl.ds(cc * 128 + u * 16, 16)
                        r1s[j, sl] = a1 * r1s[j, sl] + a2 * r2s[j, sl]
                    return 0

                lax.fori_loop(0, D // 128, col, 0)
            pltpu.async_copy(r1s, out_hbm.at[pl.ds(base0 + c * CH_C, CH_C)],
                             ss)

        issue(0, 0)

        def body(p, _):
            c0 = 2 * p
            c1 = c0 + 1
            issue(c1, 1)
            wv1 = w1_v[pl.ds(p * 16, 16)]
            wv2 = w2_v[pl.ds(p * 16, 16)]

            @pl.when(p > 0)
            def _():
                drain_store(c0 - 2, 0)

            drain_gather(c0, 0)
            compute(c0, 0, wv1, wv2, 0)

            @pl.when(c0 + 2 < nch)
            def _():
                issue(c0 + 2, 0)

            @pl.when(p > 0)
            def _():
                drain_store(c1 - 2, 1)

            drain_gather(c1, 1)
            compute(c1, 1, wv1, wv2, CH_C)
            return 0

        lax.fori_loop(0, npair, body, 0)
        drain_store(nch - 2, 0)
        drain_store(nch - 1, 1)

    return combine


def _combine(ys, pos1f, pos2f, w1f, w2f):
    return _make_combine()(ys, pos1f, pos2f, w1f, w2f)


def kernel(hidden_states, gate_w, w_gate, w_up, w_down):
    b, s, d = hidden_states.shape
    x = hidden_states.reshape(b * s, d)
    i1o, i2o, w1o, w2o = _router(x, gate_w)
    pos1, pos2, ge = _plan(i1o.reshape(NB, TB), i2o.reshape(NB, TB))
    pos1f = pos1.reshape(N)
    pos2f = pos2.reshape(N)
    xs = _dispatch(x, pos1f, pos2f)
    ys = _mm(ge.reshape(NTP), xs, w_gate, w_up, w_down)
    out = _combine(ys, pos1f, pos2f, w1o.reshape(N), w2o.reshape(N))
    return out.reshape(b, s, d)
